# baseline (speedup 1.0000x reference)
import functools

import jax
import jax.numpy as jnp
from jax import lax
from jax.experimental import pallas as pl
from jax.experimental.pallas import tpu as pltpu
from jax.experimental.pallas import tpu_sc as plsc

_N = 1000
_S = 32
_M = _N * _N
_B = 32768
_GRID = (_M + _B - 1) // _B
_TAIL = _M - (_GRID - 1) * _B
_BIG = 2**30
_MARGIN = 1e-3

_TC_ROWS = 24
_SC_ROWS = _S - _TC_ROWS
_NW = 32
_TPT = 244
_CW = 1024
_NCH = 30
_TTAIL = (_TPT - 30 * 8) * 128
_EPI_OFF = _NW * _TPT * 128
_EPI = _M - _EPI_OFF


def _scores(u, lg):
    uc = jnp.clip(u, 1e-06, 1.0 - 1e-06)
    return lg - jnp.log(-jnp.log(uc))


def _brute_body(rows, emit_lmax, u_ref, l_ref, x_ref, y_ref, lmax_ref,
                accv_ref, acci_ref, lm_ref):
    i = pl.program_id(0)
    u = u_ref[...]
    lg = l_ref[...]

    @pl.when(i == 0)
    def _init():
        accv_ref[...] = _scores(u, lg)
        acci_ref[...] = jnp.zeros((rows, _B), jnp.int32)
        if emit_lmax:
            lm_ref[0, 0] = jnp.max(lg)

    @pl.when(jnp.logical_and(i > 0, i < _GRID - 1))
    def _mid():
        s = _scores(u, lg)
        upd = s > accv_ref[...]
        accv_ref[...] = jnp.where(upd, s, accv_ref[...])
        acci_ref[...] = jnp.where(upd, i, acci_ref[...])
        if emit_lmax:
            lm_ref[0, 0] = jnp.maximum(lm_ref[0, 0], jnp.max(lg))

    @pl.when(i == _GRID - 1)
    def _last():
        col = jax.lax.broadcasted_iota(jnp.int32, (rows, _B), 1)
        s = jnp.where(col < _TAIL, _scores(u, lg), -jnp.inf)
        upd = s > accv_ref[...]
        av = jnp.where(upd, s, accv_ref[...])
        ai = jnp.where(upd, i, acci_ref[...])
        m = jnp.max(av, axis=1, keepdims=True)
        flat = ai * _B + col
        cand = jnp.where(av == m, flat, _BIG)
        ci = jnp.min(cand, axis=1, keepdims=True)
        x_ref[...] = ci // _N
        y_ref[...] = ci - (ci // _N) * _N
        if emit_lmax:
            col1 = jax.lax.broadcasted_iota(jnp.int32, (1, _B), 1)
            lgm = jnp.where(col1 < _TAIL, lg, -jnp.inf)
            lmax_ref[0, 0] = jnp.maximum(lm_ref[0, 0], jnp.max(lgm))


def _brute(uniform, lflat, rows, row_block, emit_lmax):
    body = functools.partial(_brute_body, rows, emit_lmax)
    out = pl.pallas_call(
        body,
        grid=(_GRID,),
        in_specs=[
            pl.BlockSpec((rows, _B), lambda i: (row_block, i)),
            pl.BlockSpec((1, _B), lambda i: (0, i)),
        ],
        out_specs=[
            pl.BlockSpec((rows, 1), lambda i: (0, 0)),
            pl.BlockSpec((rows, 1), lambda i: (0, 0)),
            pl.BlockSpec(memory_space=pltpu.SMEM),
        ],
        out_shape=[
            jax.ShapeDtypeStruct((rows, 1), jnp.int32),
            jax.ShapeDtypeStruct((rows, 1), jnp.int32),
            jax.ShapeDtypeStruct((1, 1), jnp.float32),
        ],
        scratch_shapes=[
            pltpu.VMEM((rows, _B), jnp.float32),
            pltpu.VMEM((rows, _B), jnp.int32),
            pltpu.SMEM((1, 1), jnp.float32),
        ],
    )(uniform, lflat)
    return out


def _sc_candidates(uniform):
    mesh = plsc.VectorSubcoreMesh(core_axis_name="c", subcore_axis_name="s")

    @functools.partial(
        pl.kernel,
        mesh=mesh,
        out_type=[
            jax.ShapeDtypeStruct((_NW, _SC_ROWS, 16), jnp.float32),
            jax.ShapeDtypeStruct((_NW, _SC_ROWS, 16), jnp.int32),
            jax.ShapeDtypeStruct((_NW, _SC_ROWS, 16), jnp.float32),
        ],
        scratch_types=[
            pltpu.VMEM((_SC_ROWS, _CW), jnp.float32),
            pltpu.VMEM((_SC_ROWS, _CW), jnp.float32),
            pltpu.VMEM((_SC_ROWS, _EPI), jnp.float32),
            pltpu.VMEM((_SC_ROWS, 16), jnp.float32),
            pltpu.VMEM((_SC_ROWS, 16), jnp.int32),
            pltpu.VMEM((_SC_ROWS, 16), jnp.float32),
            pltpu.SemaphoreType.DMA,
            pltpu.SemaphoreType.DMA,
        ],
    )
    def k(u_hbm, m1_hbm, i1_hbm, m2_hbm, buf0, buf1, ebuf, m1v, i1v, m2v,
          sem0, sem1):
        wid = lax.axis_index("s") * 2 + lax.axis_index("c")
        base0 = wid * (_TPT * 128)
        iota = lax.broadcasted_iota(jnp.int32, (16,), 0)
        bufs = (buf0, buf1)
        sems = (sem0, sem1)

        def src(c):
            return u_hbm.at[pl.ds(_TC_ROWS, _SC_ROWS),
                            pl.ds(base0 + c * _CW, _CW)]

        def acc_rows(src_buf, ncols, off, carry):
            def kbody(kk, cr):
                cur = (off + kk * 16) + iota
                new = []
                for r in range(_SC_ROWS):
                    m1, i1, m2 = cr[3 * r:3 * r + 3]
                    x = src_buf[r, pl.ds(kk * 16, 16)]
                    b = x > m1
                    new += [jnp.where(b, x, m1),
                            jnp.where(b, cur, i1),
                            jnp.maximum(m2, jnp.where(b, m1, x))]
                return tuple(new)

            return lax.fori_loop(0, ncols // 16, kbody, carry, unroll=4)

        pltpu.make_async_copy(src(0), buf0, sem0).start()
        pltpu.make_async_copy(src(1), buf1, sem1).start()

        def pair_body(c, carry):
            for b in range(2):
                cc = 2 * c + b
                pltpu.make_async_copy(src(cc), bufs[b], sems[b]).wait()
                carry = acc_rows(bufs[b], _CW, base0 + cc * _CW, carry)
                pltpu.make_async_copy(src(cc + 2), bufs[b], sems[b]).start()
            return carry

        init = []
        for _ in range(_SC_ROWS):
            init += [jnp.full((16,), -jnp.inf, jnp.float32),
                     jnp.zeros((16,), jnp.int32),
                     jnp.full((16,), -jnp.inf, jnp.float32)]
        carry = lax.fori_loop(0, _NCH // 2 - 1, pair_body, tuple(init))
        for b in range(2):
            cc = _NCH - 2 + b
            pltpu.make_async_copy(src(cc), bufs[b], sems[b]).wait()
            carry = acc_rows(bufs[b], _CW, base0 + cc * _CW, carry)

        toff = base0 + _NCH * _CW
        pltpu.sync_copy(
            u_hbm.at[pl.ds(_TC_ROWS, _SC_ROWS), pl.ds(toff, _TTAIL)],
            ebuf.at[:, pl.ds(0, _TTAIL)])
        carry = acc_rows(ebuf, _TTAIL, toff, carry)

        pltpu.sync_copy(
            u_hbm.at[pl.ds(_TC_ROWS, _SC_ROWS), pl.ds(_EPI_OFF, _EPI)], ebuf)
        carry = acc_rows(ebuf, _EPI, _EPI_OFF, carry)

        for r in range(_SC_ROWS):
            m1v[r, :] = carry[3 * r]
            i1v[r, :] = carry[3 * r + 1]
            m2v[r, :] = carry[3 * r + 2]
        pltpu.sync_copy(m1v, m1_hbm.at[wid])
        pltpu.sync_copy(i1v, i1_hbm.at[wid])
        pltpu.sync_copy(m2v, m2_hbm.at[wid])

    return k(uniform)


def _cert_body(u_ref, id_ref, lg_ref, m2_ref, lmax_ref, x_ref, y_ref, ok_ref):
    s = _scores(u_ref[...], lg_ref[...])
    m = jnp.max(s, axis=1, keepdims=True)
    cand = jnp.where(s == m, id_ref[...], _BIG)
    ci = jnp.min(cand, axis=1, keepdims=True)
    mm2 = jnp.clip(jnp.max(m2_ref[...], axis=1, keepdims=True),
                   1e-06, 1.0 - 1e-06)
    bound = lmax_ref[0, 0] - jnp.log(-jnp.log(mm2)) + _MARGIN
    ok_ref[...] = (m >= bound).astype(jnp.int32)
    x_ref[...] = ci // _N
    y_ref[...] = ci - (ci // _N) * _N


def _cert(u_cand, idx_cand, lg_cand, m2_cand, lmax):
    nc = _NW * 16
    return pl.pallas_call(
        _cert_body,
        in_specs=[
            pl.BlockSpec((_SC_ROWS, nc), lambda: (0, 0)),
            pl.BlockSpec((_SC_ROWS, nc), lambda: (0, 0)),
            pl.BlockSpec((_SC_ROWS, nc), lambda: (0, 0)),
            pl.BlockSpec((_SC_ROWS, nc), lambda: (0, 0)),
            pl.BlockSpec(memory_space=pltpu.SMEM),
        ],
        out_shape=[
            jax.ShapeDtypeStruct((_SC_ROWS, 1), jnp.int32),
            jax.ShapeDtypeStruct((_SC_ROWS, 1), jnp.int32),
            jax.ShapeDtypeStruct((_SC_ROWS, 1), jnp.int32),
        ],
    )(u_cand, idx_cand, lg_cand, m2_cand, lmax)


def kernel(uniform, logits):
    lflat = logits.reshape(1, _M)

    x24, y24, lmax = _brute(uniform, lflat, _TC_ROWS, 0, True)
    m1, i1, m2 = _sc_candidates(uniform)
    nc = _NW * 16
    m1r = m1.transpose(1, 0, 2).reshape(_SC_ROWS, nc)
    i1r = i1.transpose(1, 0, 2).reshape(_SC_ROWS, nc)
    m2r = m2.transpose(1, 0, 2).reshape(_SC_ROWS, nc)
    lg_cand = jnp.take(lflat.reshape(_M), i1r.reshape(-1),
                       mode="clip").reshape(_SC_ROWS, nc)
    x8c, y8c, ok = _cert(m1r, i1r, lg_cand, m2r, lmax)

    def _fast(_):
        return x8c, y8c

    def _slow(_):
        x8b, y8b, _ = _brute(uniform, lflat, _SC_ROWS, _TC_ROWS // _SC_ROWS,
                             False)
        return x8b, y8b

    x8, y8 = lax.cond(jnp.all(ok == 1), _fast, _slow, None)
    x = jnp.concatenate([x24.reshape(_TC_ROWS), x8.reshape(_SC_ROWS)])
    y = jnp.concatenate([y24.reshape(_TC_ROWS), y8.reshape(_SC_ROWS)])
    return x, y


# speedup vs baseline: 1.6008x; 1.6008x over previous
import jax
import jax.numpy as jnp
from jax.experimental import pallas as pl
from jax.experimental.pallas import tpu as pltpu

_N = 1000
_S = 32
_M = _N * _N
_B = 32768
_GRID = (_M + _B - 1) // _B
_TAIL = _M - (_GRID - 1) * _B
_BIG = 2**30


def _scores(u, lg):
    uc = jnp.clip(u, 1e-06, 1.0 - 1e-06)
    return lg - jnp.log(-jnp.log(uc))


def _body(u_ref, l_ref, x_ref, y_ref, accv_ref, acci_ref):
    i = pl.program_id(0)
    u = u_ref[...]
    lg = l_ref[...]

    @pl.when(i == 0)
    def _init():
        accv_ref[...] = _scores(u, lg)
        acci_ref[...] = jnp.zeros((_S, _B), jnp.int32)

    @pl.when(jnp.logical_and(i > 0, i < _GRID - 1))
    def _mid():
        s = _scores(u, lg)
        upd = s > accv_ref[...]
        accv_ref[...] = jnp.where(upd, s, accv_ref[...])
        acci_ref[...] = jnp.where(upd, i, acci_ref[...])

    @pl.when(i == _GRID - 1)
    def _last():
        col = jax.lax.broadcasted_iota(jnp.int32, (_S, _B), 1)
        s = jnp.where(col < _TAIL, _scores(u, lg), -jnp.inf)
        upd = s > accv_ref[...]
        av = jnp.where(upd, s, accv_ref[...])
        ai = jnp.where(upd, i, acci_ref[...])
        m = jnp.max(av, axis=1, keepdims=True)
        flat = ai * _B + col
        cand = jnp.where(av == m, flat, _BIG)
        ci = jnp.min(cand, axis=1, keepdims=True)
        x_ref[...] = ci // _N
        y_ref[...] = ci - (ci // _N) * _N


def kernel(uniform, logits):
    lflat = logits.reshape(1, _M)
    x2, y2 = pl.pallas_call(
        _body,
        grid=(_GRID,),
        in_specs=[
            pl.BlockSpec((_S, _B), lambda i: (0, i)),
            pl.BlockSpec((1, _B), lambda i: (0, i)),
        ],
        out_specs=[
            pl.BlockSpec((_S, 1), lambda i: (0, 0)),
            pl.BlockSpec((_S, 1), lambda i: (0, 0)),
        ],
        out_shape=[
            jax.ShapeDtypeStruct((_S, 1), jnp.int32),
            jax.ShapeDtypeStruct((_S, 1), jnp.int32),
        ],
        scratch_shapes=[
            pltpu.VMEM((_S, _B), jnp.float32),
            pltpu.VMEM((_S, _B), jnp.int32),
        ],
    )(uniform, lflat)
    return x2.reshape(_S), y2.reshape(_S)
